# Initial kernel scaffold; baseline (speedup 1.0000x reference)
import jax
import jax.numpy as jnp
from jax.experimental import pallas as pl


def kernel(item_id, tmdb_genres, release_year, avg_rating, revenue, item_table, genre_table, W_cont, b_cont, W1, b1, ln_gamma, ln_beta, W2, b2):
    raise NotImplementedError("write your pallas kernel here")


# speedup vs baseline: 1.7694x; 1.7694x over previous
import functools

import jax
import jax.numpy as jnp
from jax import lax
from jax.experimental import pallas as pl
from jax.experimental.pallas import tpu as pltpu
from jax.experimental.pallas import tpu_sc as plsc

B = 4096
D = 64
GENRE_VOCAB = 32
N_GENRES = 8


_NC, _NS = 2, 16
_NW = _NC * _NS
_BPW = B // _NW

@functools.cache
def _sc_gather_fn():
    mesh = plsc.VectorSubcoreMesh(core_axis_name="c", subcore_axis_name="s")

    @functools.partial(
        pl.kernel,
        out_type=jax.ShapeDtypeStruct((B, D), jnp.float32),
        mesh=mesh,
        scratch_types=[
            pltpu.VMEM((_BPW,), jnp.int32),
            pltpu.VMEM((_BPW, D), jnp.float32),
            pltpu.SemaphoreType.DMA,
        ],
        compiler_params=pltpu.CompilerParams(use_tc_tiling_on_sc=False),
    )
    def _sc_gather(idx_hbm, table_hbm, out_hbm, idx_v, rows_v, sem):
        wid = lax.axis_index("s") * _NC + lax.axis_index("c")
        base = wid * _BPW
        pltpu.sync_copy(idx_hbm.at[pl.ds(base, _BPW)], idx_v)
        pltpu.async_copy(table_hbm.at[idx_v], rows_v, sem).wait()
        pltpu.sync_copy(rows_v, out_hbm.at[pl.ds(base, _BPW)])

    return _sc_gather


_BLK = 512


def _tower_body(i_emb_ref, genres_ref, cont_ref, gtab_ref, wc_ref, bc_ref,
                w1_ref, b1_ref, gam_ref, bet_ref, w2_ref, b2_ref, out_ref):
    f32 = jnp.float32
    genres = genres_ref[...]
    vocab_ids = lax.broadcasted_iota(jnp.int32, (1, GENRE_VOCAB), 1)
    counts = jnp.zeros((_BLK, GENRE_VOCAB), f32)
    for g in range(N_GENRES):
        col = genres[:, g:g + 1]
        counts = counts + ((col == vocab_ids) & (col > 0)).astype(f32)
    g_sum = jnp.dot(counts, gtab_ref[...], preferred_element_type=f32)
    denom = jnp.sum(counts, axis=1, keepdims=True) + 1e-8
    g_emb = g_sum / denom

    cont_emb = jnp.maximum(
        jnp.dot(cont_ref[...], wc_ref[...], preferred_element_type=f32)
        + bc_ref[...], 0.0)

    concat = jnp.concatenate([i_emb_ref[...], g_emb, cont_emb], axis=1)
    h = jnp.dot(concat, w1_ref[...], preferred_element_type=f32) + b1_ref[...]
    mu = jnp.mean(h, axis=-1, keepdims=True)
    var = jnp.mean((h - mu) ** 2, axis=-1, keepdims=True)
    h = (h - mu) / jnp.sqrt(var + 1e-5) * gam_ref[...] + bet_ref[...]
    h = jnp.maximum(h, 0.0)
    out = jnp.dot(h, w2_ref[...], preferred_element_type=f32) + b2_ref[...]
    norm = jnp.sqrt(jnp.sum(out * out, axis=1, keepdims=True))
    out_ref[...] = out / jnp.maximum(norm, 1e-12)


def _tower(i_emb, genres, cont, gtab, wc, bc, w1, b1, gam, bet, w2, b2):
    fixed = lambda *_: (0, 0)
    row = lambda i: (i, 0)
    return pl.pallas_call(
        _tower_body,
        grid=(B // _BLK,),
        in_specs=[
            pl.BlockSpec((_BLK, D), row),
            pl.BlockSpec((_BLK, N_GENRES), row),
            pl.BlockSpec((_BLK, 3), row),
            pl.BlockSpec((GENRE_VOCAB, D), fixed),
            pl.BlockSpec((3, D), fixed),
            pl.BlockSpec((1, D), fixed),
            pl.BlockSpec((3 * D, 2 * D), fixed),
            pl.BlockSpec((1, 2 * D), fixed),
            pl.BlockSpec((1, 2 * D), fixed),
            pl.BlockSpec((1, 2 * D), fixed),
            pl.BlockSpec((2 * D, D), fixed),
            pl.BlockSpec((1, D), fixed),
        ],
        out_specs=pl.BlockSpec((_BLK, D), row),
        out_shape=jax.ShapeDtypeStruct((B, D), jnp.float32),
        compiler_params=pltpu.CompilerParams(
            dimension_semantics=("arbitrary",)),
    )(i_emb, genres, cont, gtab, wc, bc, w1, b1, gam, bet, w2, b2)


def kernel(item_id, tmdb_genres, release_year, avg_rating, revenue,
           item_table, genre_table, W_cont, b_cont, W1, b1,
           ln_gamma, ln_beta, W2, b2):
    i_emb = _sc_gather_fn()(item_id.astype(jnp.int32), item_table)
    cont = jnp.stack([release_year, avg_rating, revenue], axis=1)
    return _tower(i_emb, tmdb_genres.astype(jnp.int32), cont,
                  genre_table, W_cont, b_cont.reshape(1, D), W1,
                  b1.reshape(1, 2 * D), ln_gamma.reshape(1, 2 * D),
                  ln_beta.reshape(1, 2 * D), W2, b2.reshape(1, D))
